# Initial kernel scaffold; baseline (speedup 1.0000x reference)
import jax
import jax.numpy as jnp
from jax.experimental import pallas as pl


def kernel(x, table, bias):
    raise NotImplementedError("write your pallas kernel here")


# speedup vs baseline: 1.3503x; 1.3503x over previous
import functools

import numpy as np
import jax
import jax.numpy as jnp
from jax import lax
from jax.experimental import pallas as pl
from jax.experimental.pallas import tpu as pltpu
from jax.experimental.pallas import tpu_sc as plsc

_NUM_FIELDS = 26
_FIELD_SIZE = 38462
_BATCH = 16384
_NC, _NS, _LANES = 2, 16, 16
_NW = _NC * _NS
_RPW = _BATCH // _NW
_IPW = _RPW * _NUM_FIELDS
_CHUNK = 128
_NCHUNK = _IPW // _CHUNK
_CPF = _RPW // _CHUNK

_mesh = plsc.VectorSubcoreMesh(
    core_axis_name="c", subcore_axis_name="s",
    num_cores=_NC, num_subcores=_NS,
)


@functools.partial(
    pl.kernel,
    out_type=jax.ShapeDtypeStruct((_NW, _RPW), jnp.float32),
    mesh=_mesh,
    scratch_types=[
        pltpu.VMEM((_NCHUNK, _CHUNK), jnp.int32),
        pltpu.VMEM((_NCHUNK, _CHUNK), jnp.int32),
        pltpu.VMEM((_NUM_FIELDS, _RPW), jnp.float32),
        pltpu.VMEM((_RPW,), jnp.float32),
        pltpu.VMEM((_LANES,), jnp.float32),
        pltpu.SemaphoreType.DMA,
    ],
)
def _features_linear_sc(x_hbm, off_hbm, table_hbm, bias_hbm, out_hbm,
                        idxw, offw, valw, acc, biasw, gsem):
    wid = lax.axis_index("s") * _NC + lax.axis_index("c")

    pltpu.sync_copy(x_hbm.at[wid], idxw)
    pltpu.sync_copy(off_hbm, offw)
    pltpu.sync_copy(bias_hbm, biasw)

    def _off_body(c, carry):
        for t in range(_CHUNK // _LANES):
            sl = pl.ds(t * _LANES, _LANES)
            idxw[c, sl] = idxw[c, sl] + offw[c, sl]
        return carry
    lax.fori_loop(0, _NCHUNK, _off_body, 0)

    def _gfire(c, carry):
        f = c >> 2
        col = (c & 3) * _CHUNK
        pltpu.make_async_copy(table_hbm.at[idxw.at[c]],
                              valw.at[f, pl.ds(col, _CHUNK)], gsem).start()
        return carry
    lax.fori_loop(0, _NCHUNK, _gfire, 0)

    def _gdrain(c, carry):
        f = c >> 2
        col = (c & 3) * _CHUNK
        pltpu.make_async_copy(table_hbm.at[idxw.at[c]],
                              valw.at[f, pl.ds(col, _CHUNK)], gsem).wait()
        return carry
    lax.fori_loop(0, _NCHUNK, _gdrain, 0)

    def _red_body(j, carry):
        sl = pl.ds(j * _LANES, _LANES)
        v = biasw[...]
        for f in range(_NUM_FIELDS):
            v = v + valw[f, sl]
        acc[sl] = v
        return carry
    lax.fori_loop(0, _RPW // _LANES, _red_body, 0)

    pltpu.sync_copy(acc, out_hbm.at[wid])


def kernel(x, table, bias):
    offsets = (np.arange(_NUM_FIELDS, dtype=np.int64) * _FIELD_SIZE).astype(
        np.int32)
    off_map = np.repeat(offsets, _RPW).reshape(_NCHUNK, _CHUNK)

    x_w = x.reshape(_NW, _RPW, _NUM_FIELDS).transpose(0, 2, 1).reshape(
        _NW, _NCHUNK, _CHUNK)
    table_flat = table.reshape(-1)
    bias_v = jnp.broadcast_to(bias.astype(jnp.float32), (_LANES,))

    out = _features_linear_sc(x_w, jnp.asarray(off_map), table_flat, bias_v)
    return out.reshape(_BATCH, 1)


# speedup vs baseline: 2.5648x; 1.8995x over previous
import functools

import numpy as np
import jax
import jax.numpy as jnp
from jax import lax
from jax.experimental import pallas as pl
from jax.experimental.pallas import tpu as pltpu
from jax.experimental.pallas import tpu_sc as plsc

_NUM_FIELDS = 26
_FIELD_SIZE = 38462
_BATCH = 16384
_NC, _NS, _LANES = 2, 16, 16
_NW = _NC * _NS
_RPW = _BATCH // _NW
_IPW = _RPW * _NUM_FIELDS
_CHUNK = 128
_NCHUNK = _IPW // _CHUNK

_mesh = plsc.VectorSubcoreMesh(
    core_axis_name="c", subcore_axis_name="s",
    num_cores=_NC, num_subcores=_NS,
)


@functools.partial(
    pl.kernel,
    out_type=jax.ShapeDtypeStruct((_NW, _RPW), jnp.float32),
    mesh=_mesh,
    scratch_types=[
        pltpu.VMEM((_NCHUNK, _CHUNK), jnp.int32),
        pltpu.VMEM((_NCHUNK, _CHUNK), jnp.int32),
        pltpu.VMEM((_NUM_FIELDS, _RPW), jnp.float32),
        pltpu.VMEM((_RPW,), jnp.float32),
        pltpu.VMEM((_LANES,), jnp.float32),
        pltpu.SemaphoreType.DMA,
    ],
)
def _features_linear_sc(x_hbm, off_hbm, table_hbm, bias_hbm, out_hbm,
                        idxw, offw, valw, acc, biasw, gsem):
    wid = lax.axis_index("s") * _NC + lax.axis_index("c")
    table1d = table_hbm.at[0]

    pltpu.sync_copy(x_hbm.at[wid], idxw)
    pltpu.sync_copy(off_hbm, offw)
    pltpu.sync_copy(bias_hbm, biasw)

    def _off_body(c, carry):
        for t in range(_CHUNK // _LANES):
            sl = pl.ds(t * _LANES, _LANES)
            idxw[c, sl] = idxw[c, sl] + offw[c, sl]
        return carry
    lax.fori_loop(0, _NCHUNK, _off_body, 0)

    def _gfire(c, carry):
        f = c >> 2
        col = (c & 3) * _CHUNK
        pltpu.make_async_copy(table1d.at[idxw.at[c]],
                              valw.at[f, pl.ds(col, _CHUNK)], gsem).start()
        return carry
    lax.fori_loop(0, _NCHUNK, _gfire, 0)

    def _gdrain(c, carry):
        f = c >> 2
        col = (c & 3) * _CHUNK
        pltpu.make_async_copy(table1d.at[idxw.at[c]],
                              valw.at[f, pl.ds(col, _CHUNK)], gsem).wait()
        return carry
    lax.fori_loop(0, _NCHUNK, _gdrain, 0)

    def _red_body(j, carry):
        sl = pl.ds(j * _LANES, _LANES)
        v = biasw[...]
        for f in range(_NUM_FIELDS):
            v = v + valw[f, sl]
        acc[sl] = v
        return carry
    lax.fori_loop(0, _RPW // _LANES, _red_body, 0)

    pltpu.sync_copy(acc, out_hbm.at[wid])


def kernel(x, table, bias):
    offsets = (np.arange(_NUM_FIELDS, dtype=np.int64) * _FIELD_SIZE).astype(
        np.int32)
    off_map = np.repeat(offsets, _RPW).reshape(_NCHUNK, _CHUNK)

    x_w = x.reshape(_NW, _RPW, _NUM_FIELDS).transpose(0, 2, 1).reshape(
        _NW, _NCHUNK, _CHUNK)
    table_t = table.T
    bias_v = jnp.broadcast_to(bias.astype(jnp.float32), (_LANES,))

    out = _features_linear_sc(x_w, jnp.asarray(off_map), table_t, bias_v)
    return out.reshape(_BATCH, 1)


# speedup vs baseline: 2.7957x; 1.0900x over previous
import functools

import jax
import jax.numpy as jnp
from jax import lax
from jax.experimental import pallas as pl
from jax.experimental.pallas import tpu as pltpu
from jax.experimental.pallas import tpu_sc as plsc

_NUM_FIELDS = 26
_FIELD_SIZE = 38462
_BATCH = 16384
_NC, _NS, _LANES = 2, 16, 16
_NW = _NC * _NS
_RPW = _BATCH // _NW
_IPW = _RPW * _NUM_FIELDS
_CHUNK = 128
_NCHUNK = _IPW // _CHUNK

_mesh = plsc.VectorSubcoreMesh(
    core_axis_name="c", subcore_axis_name="s",
    num_cores=_NC, num_subcores=_NS,
)


@functools.partial(
    pl.kernel,
    out_type=jax.ShapeDtypeStruct((_NW, _RPW), jnp.float32),
    mesh=_mesh,
    scratch_types=[
        pltpu.VMEM((_NCHUNK, _CHUNK), jnp.int32),
        pltpu.VMEM((_NUM_FIELDS, _RPW), jnp.float32),
        pltpu.VMEM((_RPW,), jnp.float32),
        pltpu.VMEM((_LANES,), jnp.float32),
        pltpu.SemaphoreType.DMA,
    ],
)
def _features_linear_sc(x_hbm, table_hbm, bias_hbm, out_hbm,
                        idxw, valw, acc, biasw, gsem):
    wid = lax.axis_index("s") * _NC + lax.axis_index("c")
    table1d = table_hbm.at[0]

    pltpu.sync_copy(x_hbm.at[wid], idxw)
    pltpu.sync_copy(bias_hbm, biasw)

    def _gfire(c, carry):
        f = c >> 2
        col = (c & 3) * _CHUNK
        offv = jnp.zeros((_LANES,), jnp.int32) + f * _FIELD_SIZE
        for t in range(_CHUNK // _LANES):
            sl = pl.ds(t * _LANES, _LANES)
            idxw[c, sl] = idxw[c, sl] + offv
        pltpu.make_async_copy(table1d.at[idxw.at[c]],
                              valw.at[f, pl.ds(col, _CHUNK)], gsem).start()
        return carry
    lax.fori_loop(0, _NCHUNK, _gfire, 0)

    def _gdrain(c, carry):
        f = c >> 2
        col = (c & 3) * _CHUNK
        pltpu.make_async_copy(table1d.at[idxw.at[c]],
                              valw.at[f, pl.ds(col, _CHUNK)], gsem).wait()
        return carry
    lax.fori_loop(0, _NCHUNK, _gdrain, 0)

    def _red_body(j, carry):
        sl = pl.ds(j * _LANES, _LANES)
        v = biasw[...]
        for f in range(_NUM_FIELDS):
            v = v + valw[f, sl]
        acc[sl] = v
        return carry
    lax.fori_loop(0, _RPW // _LANES, _red_body, 0)

    pltpu.sync_copy(acc, out_hbm.at[wid])


def kernel(x, table, bias):
    x_w = x.reshape(_NW, _RPW, _NUM_FIELDS).transpose(0, 2, 1).reshape(
        _NW, _NCHUNK, _CHUNK)
    table_t = table.T
    bias_v = jnp.broadcast_to(bias.astype(jnp.float32), (_LANES,))

    out = _features_linear_sc(x_w, table_t, bias_v)
    return out.reshape(_BATCH, 1)


# speedup vs baseline: 3.2000x; 1.1446x over previous
import functools

import jax
import jax.numpy as jnp
from jax import lax
from jax.experimental import pallas as pl
from jax.experimental.pallas import tpu as pltpu
from jax.experimental.pallas import tpu_sc as plsc

_NUM_FIELDS = 26
_FIELD_SIZE = 38462
_BATCH = 16384
_NC, _NS, _LANES = 2, 16, 16
_NW = _NC * _NS
_RPW = _BATCH // _NW
_IPW = _RPW * _NUM_FIELDS
_CHUNK = 128
_NCHUNK = _IPW // _CHUNK

_mesh = plsc.VectorSubcoreMesh(
    core_axis_name="c", subcore_axis_name="s",
    num_cores=_NC, num_subcores=_NS,
)


@functools.partial(
    pl.kernel,
    out_type=jax.ShapeDtypeStruct((1, _BATCH), jnp.float32),
    mesh=_mesh,
    scratch_types=[
        pltpu.VMEM((_NUM_FIELDS, _RPW), jnp.int32),
        pltpu.VMEM((_NCHUNK, _CHUNK), jnp.int32),
        pltpu.VMEM((_NUM_FIELDS, _RPW), jnp.float32),
        pltpu.VMEM((_RPW,), jnp.float32),
        pltpu.VMEM((_LANES,), jnp.float32),
        pltpu.SemaphoreType.DMA,
    ],
)
def _features_linear_sc(x_hbm, table_hbm, bias_hbm, out_hbm,
                        xw, idxw, valw, acc, biasw, gsem):
    wid = lax.axis_index("s") * _NC + lax.axis_index("c")
    table1d = table_hbm.at[0]

    pltpu.sync_copy(x_hbm.at[:, pl.ds(wid * _RPW, _RPW)], xw)
    pltpu.sync_copy(bias_hbm, biasw.at[pl.ds(0, 1)])

    def _gfire(c, carry):
        f = c >> 2
        col = (c & 3) * _CHUNK
        offv = jnp.zeros((_LANES,), jnp.int32) + f * _FIELD_SIZE
        for t in range(_CHUNK // _LANES):
            sl = pl.ds(t * _LANES, _LANES)
            idxw[c, sl] = xw[f, pl.ds(col + t * _LANES, _LANES)] + offv
        pltpu.make_async_copy(table1d.at[idxw.at[c]],
                              valw.at[f, pl.ds(col, _CHUNK)], gsem).start()
        return carry
    lax.fori_loop(0, _NCHUNK, _gfire, 0)

    def _gdrain(c, carry):
        f = c >> 2
        col = (c & 3) * _CHUNK
        pltpu.make_async_copy(table1d.at[idxw.at[c]],
                              valw.at[f, pl.ds(col, _CHUNK)], gsem).wait()
        return carry
    lax.fori_loop(0, _NCHUNK, _gdrain, 0)

    bias_vec = jnp.zeros((_LANES,), jnp.float32) + biasw[...][0]

    def _red_body(j, carry):
        sl = pl.ds(j * _LANES, _LANES)
        v = bias_vec
        for f in range(_NUM_FIELDS):
            v = v + valw[f, sl]
        acc[sl] = v
        return carry
    lax.fori_loop(0, _RPW // _LANES, _red_body, 0)

    pltpu.sync_copy(acc, out_hbm.at[0, pl.ds(wid * _RPW, _RPW)])


def kernel(x, table, bias):
    x_t = x.T
    table_t = table.T

    out = _features_linear_sc(x_t, table_t, bias.astype(jnp.float32))
    return out.reshape(_BATCH, 1)
